# baseline (speedup 1.0000x reference)
import functools

import jax
import jax.numpy as jnp
from jax.experimental import pallas as pl
from jax.experimental.pallas import tpu as pltpu

_K = 384


def _kwta_body(x_ref, out_ref, gate_ref, gate_vmem, *, k, ttile):
    bb, t, c = x_ref.shape
    j = pl.program_id(1)

    @pl.when(j == 0)
    def _():
        halves = []
        ch = c // 2
        for ci in range(2):
            def reduce_chunk(jj, acc, ci=ci):
                return acc + jnp.abs(
                    x_ref[:, pl.ds(jj * 8, 8), pl.ds(ci * ch, ch)])

            acc = jax.lax.fori_loop(0, t // 8, reduce_chunk,
                                    jnp.zeros((bb, 8, ch), jnp.float32))
            halves.append(jnp.sum(acc, axis=1))
        s = jnp.concatenate(halves, axis=1) / jnp.float32(t)

        s_bits = jax.lax.bitcast_convert_type(s, jnp.int32)
        p = jnp.zeros((bb, 1), jnp.int32)
        for sh in range(28, -1, -4):
            digit = jnp.zeros((bb, 1), jnp.int32)
            for d in range(1, 8 if sh == 28 else 16):
                cand = p + jnp.int32(d << sh)
                cnt = jnp.sum((s_bits >= cand).astype(jnp.int32), axis=1,
                              keepdims=True)
                digit += (cnt >= k).astype(jnp.int32)
            p = p + (digit << sh)
        thresh = jax.lax.bitcast_convert_type(p, jnp.float32)

        gate = (s >= thresh).astype(jnp.float32)
        gate_vmem[...] = gate
        gate_ref[...] = gate[None]

    gate = gate_vmem[...]
    out_ref[...] = x_ref[:, pl.ds(j * ttile, ttile), :] * gate[:, None, :]


def kernel(x):
    b, t, c = x.shape
    bb = 8
    ttile = 96
    grid = (b // bb, t // ttile)
    out_gated, gate = pl.pallas_call(
        functools.partial(_kwta_body, k=_K, ttile=ttile),
        grid=grid,
        in_specs=[pl.BlockSpec((bb, t, c), lambda i, j: (i, 0, 0))],
        out_specs=[
            pl.BlockSpec((bb, ttile, c), lambda i, j: (i, j, 0)),
            pl.BlockSpec((1, bb, c), lambda i, j: (i, 0, 0)),
        ],
        out_shape=[
            jax.ShapeDtypeStruct((b, t, c), x.dtype),
            jax.ShapeDtypeStruct((b // bb, bb, c), x.dtype),
        ],
        scratch_shapes=[pltpu.VMEM((bb, c), jnp.float32)],
        compiler_params=pltpu.CompilerParams(
            dimension_semantics=("parallel", "arbitrary"),
        ),
    )(x)
    return (out_gated, gate.reshape(b, c))


# speedup vs baseline: 1.4218x; 1.4218x over previous
import functools

import jax
import jax.numpy as jnp
from jax.experimental import pallas as pl
from jax.experimental.pallas import tpu as pltpu

_K = 384

_TCHUNK = 64


def _kwta_body(x_ref, out_ref, gate_ref, *, k):
    bb, t, c = x_ref.shape
    nchunks = t // _TCHUNK

    halves = []
    ch = c // 2
    for ci in range(2):
        def reduce_chunk(j, acc, ci=ci):
            return acc + jnp.abs(
                x_ref[:, pl.ds(j * 8, 8), pl.ds(ci * ch, ch)])

        acc = jax.lax.fori_loop(0, t // 8, reduce_chunk,
                                jnp.zeros((bb, 8, ch), jnp.float32))
        halves.append(jnp.sum(acc, axis=1))
    s = jnp.concatenate(halves, axis=1) / jnp.float32(t)

    s_bits = jax.lax.bitcast_convert_type(s, jnp.int32)
    p = jnp.zeros((bb, 1), jnp.int32)
    for sh in range(28, -1, -4):
        digit = jnp.zeros((bb, 1), jnp.int32)
        for d in range(1, 8 if sh == 28 else 16):
            cand = p + jnp.int32(d << sh)
            cnt = jnp.sum((s_bits >= cand).astype(jnp.int32), axis=1,
                          keepdims=True)
            digit += (cnt >= k).astype(jnp.int32)
        p = p + (digit << sh)
    thresh = jax.lax.bitcast_convert_type(p, jnp.float32)

    gate = (s >= thresh).astype(jnp.float32)
    gate_ref[...] = gate[None]

    def gate_chunk(j, carry):
        sl = pl.ds(j * _TCHUNK, _TCHUNK)
        out_ref[:, sl, :] = x_ref[:, sl, :] * gate[:, None, :]
        return carry

    jax.lax.fori_loop(0, nchunks, gate_chunk, 0)


def kernel(x):
    b, t, c = x.shape
    bb = 8
    grid = (b // bb,)
    out_gated, gate = pl.pallas_call(
        functools.partial(_kwta_body, k=_K),
        grid=grid,
        in_specs=[pl.BlockSpec((bb, t, c), lambda i: (i, 0, 0))],
        out_specs=[
            pl.BlockSpec((bb, t, c), lambda i: (i, 0, 0)),
            pl.BlockSpec((1, bb, c), lambda i: (i, 0, 0)),
        ],
        out_shape=[
            jax.ShapeDtypeStruct((b, t, c), x.dtype),
            jax.ShapeDtypeStruct((b // bb, bb, c), x.dtype),
        ],
        compiler_params=pltpu.CompilerParams(
            dimension_semantics=("parallel",),
        ),
    )(x)
    return (out_gated, gate.reshape(b, c))
